# baseline (speedup 1.0000x reference)
import functools

import jax
import jax.numpy as jnp
from jax import lax
from jax.experimental import pallas as pl
from jax.experimental.pallas import tpu as pltpu
from jax.experimental.pallas import tpu_sc as plsc


def _make_sc_add(R, D):
    info = plsc.get_sparse_core_info()
    NC, NS = info.num_cores, info.num_subcores
    NW = NC * NS
    CH = 16
    rows_per_w = R // NW
    n_chunks = rows_per_w // CH
    vecs_per_row = D // 16
    UNROLL = 8
    mesh = plsc.VectorSubcoreMesh(core_axis_name="c", subcore_axis_name="s")

    @functools.partial(
        pl.kernel,
        out_type=jax.ShapeDtypeStruct((R, D), jnp.float32),
        mesh=mesh,
        scratch_types=[
            pltpu.VMEM((CH, D), jnp.float32),
            pltpu.VMEM((CH, D), jnp.float32),
        ],
    )
    def sc_add(x_hbm, e_hbm, out_hbm, bufx, bufe):
        wid = lax.axis_index("s") * NC + lax.axis_index("c")
        w_base = wid * rows_per_w

        def chunk_body(k, carry):
            base = w_base + k * CH
            pltpu.sync_copy(x_hbm.at[pl.ds(base, CH)], bufx)
            pltpu.sync_copy(e_hbm.at[pl.ds(base, CH)], bufe)

            def row_body(r, c2):
                def col_body(cb, c3):
                    for u in range(UNROLL):
                        col = (cb * UNROLL + u) * 16
                        plsc.addupdate(
                            bufx.at[r, pl.ds(col, 16)],
                            bufe[r, pl.ds(col, 16)],
                        )
                    return c3

                return lax.fori_loop(0, vecs_per_row // UNROLL, col_body, c2)

            lax.fori_loop(0, CH, row_body, 0)
            pltpu.sync_copy(bufx, out_hbm.at[pl.ds(base, CH)])
            return carry

        lax.fori_loop(0, n_chunks, chunk_body, 0)

    return sc_add


def kernel(x, node_pos_emb):
    B, L, D = x.shape
    R = B * L
    x2 = x.reshape(R, D)
    e2 = node_pos_emb.reshape(R, D)
    out = _make_sc_add(R, D)(x2, e2)
    return out.reshape(B, L, D)
